# baseline (speedup 1.0000x reference)
import functools

import jax
import jax.numpy as jnp
from jax import lax
from jax.experimental import pallas as pl
from jax.experimental.pallas import tpu as pltpu
from jax.experimental.pallas import tpu_sc as plsc

N_SO = 100000
DIM = 32
BATCH = 16384
N_ELEC = 50

NC = 2
NS = 16
NW = NC * NS
B_PER_W = BATCH // NW
BCHUNK = 128
NCH = B_PER_W // BCHUNK
NBUF = 4
N_STEPS = N_ELEC * NCH


def _accum_chunk(buf, out_v, base_row):
    for i in range(BCHUNK):
        for h in range(2):
            sl = pl.ds(16 * h, 16)
            plsc.addupdate(out_v.at[base_row + i, sl], buf[i, sl])


@functools.partial(
    pl.kernel,
    out_type=jax.ShapeDtypeStruct((BATCH, DIM), jnp.float32),
    mesh=plsc.VectorSubcoreMesh(core_axis_name="c", subcore_axis_name="s"),
    compiler_params=pltpu.CompilerParams(use_tc_tiling_on_sc=False),
    scratch_types=(
        [pltpu.VMEM((N_ELEC, B_PER_W), jnp.int32)]
        + [pltpu.VMEM((BCHUNK, DIM), jnp.float32)] * NBUF
        + [pltpu.VMEM((B_PER_W, DIM), jnp.float32)]
        + [pltpu.VMEM((DIM,), jnp.float32)]
        + [pltpu.SemaphoreType.DMA] * NBUF
    ),
)
def _pool_kernel(occt_hbm, e_hbm, b_hbm, out_hbm, idx_v, *rest):
    bufs = rest[:NBUF]
    out_v = rest[NBUF]
    b_v = rest[NBUF + 1]
    sems = rest[NBUF + 2:]

    wid = lax.axis_index("s") * NC + lax.axis_index("c")

    pltpu.sync_copy(b_hbm, b_v)
    pltpu.sync_copy(occt_hbm.at[:, pl.ds(wid * B_PER_W, B_PER_W)], idx_v)

    def init_body(i, carry):
        out_v[i, pl.ds(0, 16)] = b_v[pl.ds(0, 16)]
        out_v[i, pl.ds(16, 16)] = b_v[pl.ds(16, 16)]
        return carry

    lax.fori_loop(0, B_PER_W, init_body, 0)

    def idx_ref(g):
        return idx_v.at[g // NCH, pl.ds((g % NCH) * BCHUNK, BCHUNK)]

    for k in range(NBUF):
        pltpu.async_copy(e_hbm.at[idx_ref(k)], bufs[k], sems[k])

    def body(j, carry):
        g0 = j * NBUF
        for k in range(NBUF):
            g = g0 + k
            pltpu.make_async_copy(e_hbm.at[idx_ref(g)], bufs[k], sems[k]).wait()
            _accum_chunk(bufs[k], out_v, (g % NCH) * BCHUNK)

            @pl.when(g + NBUF < N_STEPS)
            def _():
                pltpu.async_copy(e_hbm.at[idx_ref(g + NBUF)], bufs[k], sems[k])

        return carry

    lax.fori_loop(0, N_STEPS // NBUF, body, 0)

    pltpu.sync_copy(out_v, out_hbm.at[pl.ds(wid * B_PER_W, B_PER_W), :])


def kernel(occ_so, E, b):
    return _pool_kernel(occ_so.astype(jnp.int32).T, E, b)


# speedup vs baseline: 1.9044x; 1.9044x over previous
import functools

import jax
import jax.numpy as jnp
from jax import lax
from jax.experimental import pallas as pl
from jax.experimental.pallas import tpu as pltpu
from jax.experimental.pallas import tpu_sc as plsc

N_SO = 100000
DIM = 32
BATCH = 16384
N_ELEC = 50

NC = 2
NS = 16
NW = NC * NS
B_PER_W = BATCH // NW
NBUF = 16


def _accum_row(buf, out_v, b_v, out_row):
    for h in range(2):
        sl = pl.ds(16 * h, 16)
        acc = [buf[k, sl] for k in range(4)]
        for e in range(4, N_ELEC):
            acc[e % 4] = acc[e % 4] + buf[e, sl]
        out_v[out_row, sl] = ((acc[0] + acc[1]) + (acc[2] + acc[3])) + b_v[sl]


@functools.partial(
    pl.kernel,
    out_type=jax.ShapeDtypeStruct((BATCH, DIM), jnp.float32),
    mesh=plsc.VectorSubcoreMesh(core_axis_name="c", subcore_axis_name="s"),
    compiler_params=pltpu.CompilerParams(use_tc_tiling_on_sc=False),
    scratch_types=(
        [pltpu.VMEM((B_PER_W, N_ELEC), jnp.int32)]
        + [pltpu.VMEM((N_ELEC, DIM), jnp.float32)] * NBUF
        + [pltpu.VMEM((B_PER_W, DIM), jnp.float32)]
        + [pltpu.VMEM((DIM,), jnp.float32)]
        + [pltpu.SemaphoreType.DMA] * NBUF
    ),
)
def _pool_kernel(occ_hbm, e_hbm, b_hbm, out_hbm, idx_v, *rest):
    bufs = rest[:NBUF]
    out_v = rest[NBUF]
    b_v = rest[NBUF + 1]
    sems = rest[NBUF + 2:]

    wid = lax.axis_index("s") * NC + lax.axis_index("c")

    pltpu.sync_copy(b_hbm, b_v)
    pltpu.sync_copy(occ_hbm.at[pl.ds(wid * B_PER_W, B_PER_W), :], idx_v)

    for k in range(NBUF):
        pltpu.async_copy(e_hbm.at[idx_v.at[k]], bufs[k], sems[k])

    def body(j, carry):
        r = j * NBUF
        for k in range(NBUF):
            pltpu.make_async_copy(e_hbm.at[idx_v.at[r + k]], bufs[k], sems[k]).wait()
            _accum_row(bufs[k], out_v, b_v, r + k)

            @pl.when(r + k + NBUF < B_PER_W)
            def _():
                pltpu.async_copy(e_hbm.at[idx_v.at[r + k + NBUF]], bufs[k], sems[k])

        return carry

    lax.fori_loop(0, B_PER_W // NBUF, body, 0)

    pltpu.sync_copy(out_v, out_hbm.at[pl.ds(wid * B_PER_W, B_PER_W), :])


def kernel(occ_so, E, b):
    return _pool_kernel(occ_so.astype(jnp.int32), E, b)


# speedup vs baseline: 2.3461x; 1.2319x over previous
import functools

import jax
import jax.numpy as jnp
from jax import lax
from jax.experimental import pallas as pl
from jax.experimental.pallas import tpu as pltpu
from jax.experimental.pallas import tpu_sc as plsc

N_SO = 100000
DIM = 32
BATCH = 16384
N_ELEC = 50

NC = 2
NS = 16
NW = NC * NS
B_PER_W = BATCH // NW
NBUF = 8


def _accum_row(buf, out_v, b_v, out_row):
    for h in range(2):
        sl = pl.ds(16 * h, 16)
        acc = [buf[k, sl] for k in range(4)]
        for e in range(4, N_ELEC):
            acc[e % 4] = acc[e % 4] + buf[e, sl]
        out_v[out_row, sl] = ((acc[0] + acc[1]) + (acc[2] + acc[3])) + b_v[sl]


@functools.partial(
    pl.kernel,
    out_type=jax.ShapeDtypeStruct((BATCH, DIM), jnp.float32),
    mesh=plsc.VectorSubcoreMesh(core_axis_name="c", subcore_axis_name="s"),
    compiler_params=pltpu.CompilerParams(use_tc_tiling_on_sc=False),
    scratch_types=(
        [pltpu.VMEM((B_PER_W, N_ELEC), jnp.int32)]
        + [pltpu.VMEM((N_ELEC, DIM), jnp.float32)] * NBUF
        + [pltpu.VMEM((B_PER_W, DIM), jnp.float32)]
        + [pltpu.VMEM((DIM,), jnp.float32)]
        + [pltpu.SemaphoreType.DMA] * NBUF
    ),
)
def _pool_kernel(occ_hbm, e_hbm, b_hbm, out_hbm, idx_v, *rest):
    bufs = rest[:NBUF]
    out_v = rest[NBUF]
    b_v = rest[NBUF + 1]
    sems = rest[NBUF + 2:]

    wid = lax.axis_index("s") * NC + lax.axis_index("c")

    pltpu.sync_copy(b_hbm, b_v)
    pltpu.sync_copy(occ_hbm.at[pl.ds(wid * B_PER_W, B_PER_W), :], idx_v)

    for k in range(NBUF):
        pltpu.async_copy(e_hbm.at[idx_v.at[k]], bufs[k], sems[k])

    def body(j, carry):
        r = j * NBUF
        for k in range(NBUF):
            pltpu.make_async_copy(e_hbm.at[idx_v.at[r + k]], bufs[k], sems[k]).wait()
            _accum_row(bufs[k], out_v, b_v, r + k)

            @pl.when(r + k + NBUF < B_PER_W)
            def _():
                pltpu.async_copy(e_hbm.at[idx_v.at[r + k + NBUF]], bufs[k], sems[k])

        return carry

    lax.fori_loop(0, B_PER_W // NBUF, body, 0)

    pltpu.sync_copy(out_v, out_hbm.at[pl.ds(wid * B_PER_W, B_PER_W), :])


def kernel(occ_so, E, b):
    return _pool_kernel(occ_so.astype(jnp.int32), E, b)
